# baseline (speedup 1.0000x reference)
import jax
import jax.numpy as jnp
from jax.experimental import pallas as pl
from jax.experimental.pallas import tpu as pltpu

_N_SECANT = 7
_CHUNK = 512
_EPS = 1e-12


def _lane_tree(u, width):
    while width > 128:
        h = width // 2
        u = u[:, :h] + u[:, h:width]
        width = h
    return u


def _entmax_block_kernel(x_ref, o_ref):
    rows, n = x_ref.shape
    nch = n // _CHUNK

    m = x_ref[:, 0:_CHUNK]
    for j in range(1, nch):
        m = jnp.maximum(m, x_ref[:, j * _CHUNK:(j + 1) * _CHUNK])
    rowmax = jnp.max(m, axis=1, keepdims=True)
    t0 = rowmax - 1.0

    zero_chunk = jnp.zeros((rows, _CHUNK), jnp.float32)

    tb = t0 + zero_chunk
    s1a = jnp.zeros((rows, 128), jnp.float32)
    s2a = jnp.zeros((rows, 128), jnp.float32)
    for j in range(nch):
        u = jnp.maximum(x_ref[:, j * _CHUNK:(j + 1) * _CHUNK] - tb, 0.0)
        uu = u * u
        s1a = s1a + _lane_tree(u, _CHUNK)
        s2a = s2a + _lane_tree(uu, _CHUNK)
    s1 = jnp.sum(s1a, axis=1, keepdims=True)
    f0 = jnp.sum(s2a, axis=1, keepdims=True)
    t1 = t0 + (f0 - 1.0) / (2.0 * jnp.maximum(s1, 1e-30))

    def secant_body(_, carry):
        t, t_prev, f_prev, tbest, fbest = carry
        tb = t + zero_chunk
        s2a = jnp.zeros((rows, 128), jnp.float32)
        for j in range(nch):
            u = jnp.maximum(x_ref[:, j * _CHUNK:(j + 1) * _CHUNK] - tb, 0.0)
            s2a = s2a + _lane_tree(u * u, _CHUNK)
        f = jnp.sum(s2a, axis=1, keepdims=True)
        ge = f >= 1.0
        tbest = jnp.where(ge, t, tbest)
        fbest = jnp.where(ge, f, fbest)
        df = f_prev - f
        df = jnp.where(jnp.abs(df) > 1e-30, df, 1e-30)
        step = jnp.clip((f - 1.0) * (t - t_prev) / df, -1.0, 1.0)
        t_new = jnp.clip(t + step, rowmax - 1.0, rowmax)
        return t_new, t, f, tbest, fbest

    _, _, _, tau, norm = jax.lax.fori_loop(
        0, _N_SECANT, secant_body, (t1, t0, f0, t0, f0)
    )

    inv = 1.0 / jnp.maximum(norm, _EPS)
    taub = tau + zero_chunk
    invb = inv + zero_chunk
    for j in range(nch):
        u = jnp.maximum(x_ref[:, j * _CHUNK:(j + 1) * _CHUNK] - taub, 0.0)
        o_ref[:, j * _CHUNK:(j + 1) * _CHUNK] = u * u * invb


def kernel(inputs):
    rows, n = inputs.shape
    block_rows = 32
    grid = (rows // block_rows,)
    return pl.pallas_call(
        _entmax_block_kernel,
        grid=grid,
        in_specs=[pl.BlockSpec((block_rows, n), lambda i: (i, 0))],
        out_specs=pl.BlockSpec((block_rows, n), lambda i: (i, 0)),
        out_shape=jax.ShapeDtypeStruct((rows, n), inputs.dtype),
        compiler_params=pltpu.CompilerParams(
            dimension_semantics=("parallel",),
        ),
    )(inputs)


# speedup vs baseline: 1.0976x; 1.0976x over previous
import jax
import jax.numpy as jnp
from jax.experimental import pallas as pl
from jax.experimental.pallas import tpu as pltpu

_N_SECANT = 6
_CHUNK = 512
_EPS = 1e-12


def _lane_tree(u, width):
    while width > 128:
        h = width // 2
        u = u[:, :h] + u[:, h:width]
        width = h
    return u


def _entmax_block_kernel(x_ref, o_ref):
    rows, n = x_ref.shape
    nch = n // _CHUNK

    m = x_ref[:, 0:_CHUNK]
    for j in range(1, nch):
        m = jnp.maximum(m, x_ref[:, j * _CHUNK:(j + 1) * _CHUNK])
    rowmax = jnp.max(m, axis=1, keepdims=True)
    t0 = rowmax - 1.0

    zero_chunk = jnp.zeros((rows, _CHUNK), jnp.float32)

    tb = t0 + zero_chunk
    s1a = jnp.zeros((rows, 128), jnp.float32)
    s2a = jnp.zeros((rows, 128), jnp.float32)
    for j in range(nch):
        u = jnp.maximum(x_ref[:, j * _CHUNK:(j + 1) * _CHUNK] - tb, 0.0)
        uu = u * u
        s1a = s1a + _lane_tree(u, _CHUNK)
        s2a = s2a + _lane_tree(uu, _CHUNK)
    s1 = jnp.sum(s1a, axis=1, keepdims=True)
    f0 = jnp.sum(s2a, axis=1, keepdims=True)
    t1 = t0 + (f0 - 1.0) / (2.0 * jnp.maximum(s1, 1e-30))

    def secant_body(_, carry):
        t, t_prev, f_prev, tbest, fbest = carry
        tb = t + zero_chunk
        s2a = jnp.zeros((rows, 128), jnp.float32)
        for j in range(nch):
            u = jnp.maximum(x_ref[:, j * _CHUNK:(j + 1) * _CHUNK] - tb, 0.0)
            s2a = s2a + _lane_tree(u * u, _CHUNK)
        f = jnp.sum(s2a, axis=1, keepdims=True)
        ge = f >= 1.0
        tbest = jnp.where(jnp.logical_and(ge, t > tbest), t, tbest)
        fbest = jnp.where(jnp.logical_and(ge, t >= tbest), f, fbest)
        df = f_prev - f
        df = jnp.where(jnp.abs(df) > 1e-30, df, 1e-30)
        step = jnp.clip((f - 1.0) * (t - t_prev) / df, -1.0, 1.0)
        t_new = jnp.clip(t + step, rowmax - 1.0, rowmax)
        return t_new, t, f, tbest, fbest

    _, _, _, tau, norm = jax.lax.fori_loop(
        0, _N_SECANT, secant_body, (t1, t0, f0, t0, f0)
    )

    inv = 1.0 / jnp.maximum(norm, _EPS)
    taub = tau + zero_chunk
    invb = inv + zero_chunk
    for j in range(nch):
        u = jnp.maximum(x_ref[:, j * _CHUNK:(j + 1) * _CHUNK] - taub, 0.0)
        o_ref[:, j * _CHUNK:(j + 1) * _CHUNK] = u * u * invb


def kernel(inputs):
    rows, n = inputs.shape
    block_rows = 32
    grid = (rows // block_rows,)
    return pl.pallas_call(
        _entmax_block_kernel,
        grid=grid,
        in_specs=[pl.BlockSpec((block_rows, n), lambda i: (i, 0))],
        out_specs=pl.BlockSpec((block_rows, n), lambda i: (i, 0)),
        out_shape=jax.ShapeDtypeStruct((rows, n), inputs.dtype),
        compiler_params=pltpu.CompilerParams(
            dimension_semantics=("parallel",),
        ),
    )(inputs)
